# Initial kernel scaffold; baseline (speedup 1.0000x reference)
import jax
import jax.numpy as jnp
from jax.experimental import pallas as pl


def kernel(user, item, genres, writers, directors, year, emb_table, fc_table, bias, W1, b1, W2, b2, W3, b3, W4, b4):
    raise NotImplementedError("write your pallas kernel here")


# speedup vs baseline: 2.5047x; 2.5047x over previous
import dataclasses
import functools

import jax
import jax.numpy as jnp
from jax import lax
from jax.experimental import pallas as pl
from jax.experimental.pallas import tpu as pltpu
from jax.experimental.pallas import tpu_sc as plsc

PAD_ID = 1500000
D = 32
LF = 20
NC, NS = 2, 16
NW = NC * NS
B_TOT = 16384
EPT = B_TOT // NW
EPO = 4
IPO = EPO * LF
OPT = EPT // EPO
NBUF = 4
UIY_OPS = EPT // 128
BLK = 2048


def _sc_compiler_params():
    cp = pltpu.CompilerParams(use_tc_tiling_on_sc=False)
    if "needs_layout_passes" in pltpu.CompilerParams.__dataclass_fields__:
        cp = dataclasses.replace(cp, needs_layout_passes=False)
    return cp


@functools.lru_cache(maxsize=None)
def _make_sc_kernel(interpret=False):
    n_fc_chunks = 3 * EPT // 16
    scratch = [
        pltpu.VMEM((OPT, IPO), jnp.int32),
        pltpu.VMEM((3 * UIY_OPS, 128), jnp.int32),
        pltpu.VMEM((3 * UIY_OPS, 128), jnp.int32),
        pltpu.VMEM((EPT, D), jnp.float32),
        pltpu.VMEM((3 * EPT, D), jnp.float32),
        pltpu.VMEM((3 * EPT, 16), jnp.float32),
        pltpu.VMEM((n_fc_chunks, 16), jnp.float32),
    ]
    scratch += [pltpu.VMEM((IPO, D), jnp.float32) for _ in range(NBUF)]
    scratch += [pltpu.SemaphoreType.DMA for _ in range(NBUF + 1)]

    def body(emb_hbm, fc_hbm, uiy_hbm, pool_hbm, feat_hbm, fcv_hbm,
             idx_v, uiy_idx_v, fcrow_idx_v, pooled_v, urows_v, fcrows_v,
             fcr2_v, rb0, rb1, rb2, rb3, s0, s1, s2, s3, su):
        rbufs = (rb0, rb1, rb2, rb3)
        sems = (s0, s1, s2, s3)
        wid = lax.axis_index("s") * NC + lax.axis_index("c")
        base = wid * EPT

        def pool_slot(rb, elem_base):
            c0, c1 = pl.ds(0, 16), pl.ds(16, 16)

            @pl.loop(0, EPO)
            def _(el):
                row0 = el * LF
                acc0 = rb.at[row0, c0][...]
                acc1 = rb.at[row0, c1][...]
                for k in range(1, LF):
                    acc0 = acc0 + rb.at[row0 + k, c0][...]
                    acc1 = acc1 + rb.at[row0 + k, c1][...]
                pooled_v.at[elem_base + el, c0][...] = acc0
                pooled_v.at[elem_base + el, c1][...] = acc1

        pltpu.sync_copy(uiy_hbm.at[pl.ds(wid * 3 * UIY_OPS, 3 * UIY_OPS)],
                        uiy_idx_v)

        @pl.loop(0, 3 * UIY_OPS)
        def _(r):
            for c in range(8):
                col = pl.ds(c * 16, 16)
                raw = uiy_idx_v.at[r, col][...]
                fcrow_idx_v.at[r, col][...] = jnp.right_shift(raw, 4)

        for r in range(3 * UIY_OPS):
            pltpu.make_async_copy(
                emb_hbm.at[uiy_idx_v.at[r]],
                urows_v.at[pl.ds(r * 128, 128)], su).start()
            pltpu.make_async_copy(
                fc_hbm.at[fcrow_idx_v.at[r]],
                fcrows_v.at[pl.ds(r * 128, 128)], su).start()

        for f in range(3):
            pltpu.sync_copy(pool_hbm.at[f, pl.ds(wid * OPT, OPT)], idx_v)
            for b in range(NBUF):
                pltpu.make_async_copy(
                    emb_hbm.at[idx_v.at[b]], rbufs[b], sems[b]).start()

            @pl.loop(0, OPT, step=NBUF)
            def _(j):
                for b in range(NBUF):
                    op = j + b
                    pltpu.make_async_copy(
                        emb_hbm.at[idx_v.at[op]], rbufs[b], sems[b]).wait()
                    pool_slot(rbufs[b], op * EPO)
                    nxt = op + NBUF

                    @pl.when(nxt < OPT)
                    def _():
                        pltpu.make_async_copy(
                            emb_hbm.at[idx_v.at[nxt]], rbufs[b], sems[b]).start()

            pltpu.sync_copy(pooled_v, feat_hbm.at[2 + f, pl.ds(base, EPT)])

        for r in range(3 * UIY_OPS):
            pltpu.make_async_copy(
                emb_hbm.at[uiy_idx_v.at[r]],
                urows_v.at[pl.ds(r * 128, 128)], su).wait()
            pltpu.make_async_copy(
                fc_hbm.at[fcrow_idx_v.at[r]],
                fcrows_v.at[pl.ds(r * 128, 128)], su).wait()

        iota16 = lax.iota(jnp.int32, 16)

        @pl.loop(0, 3 * UIY_OPS)
        def _(r):
            for c in range(8):
                col = pl.ds(c * 16, 16)
                raw = uiy_idx_v.at[r, col][...]
                lane = jnp.bitwise_and(raw, 15)
                t = r * 8 + c
                vals = plsc.load_gather(fcrows_v, [t * 16 + iota16, lane])
                fcr2_v.at[t, pl.ds(0, 16)][...] = vals

        pltpu.sync_copy(fcr2_v, fcv_hbm.at[pl.ds(wid * (3 * EPT // 16),
                                                 3 * EPT // 16)])
        for f, plane in ((0, 0), (1, 1), (2, 5)):
            pltpu.sync_copy(urows_v.at[pl.ds(f * EPT, EPT)],
                            feat_hbm.at[plane, pl.ds(base, EPT)])

    return pl.kernel(
        body,
        out_type=(jax.ShapeDtypeStruct((6, B_TOT, D), jnp.float32),
                  jax.ShapeDtypeStruct((NW * 3 * EPT // 16, 16), jnp.float32)),
        mesh=plsc.VectorSubcoreMesh(core_axis_name="c", subcore_axis_name="s",
                                    num_cores=NC, num_subcores=NS),
        scratch_types=scratch,
        compiler_params=_sc_compiler_params(),
        interpret=interpret,
    )


def _tc_body(feat_ref, g_ref, w_ref, dd_ref, fct_ref, bias_ref,
             W1_ref, b1_ref, W2_ref, b2_ref, W3_ref, b3_ref, W4_ref, b4_ref,
             out_ref):
    feat = feat_ref[...]

    def mean(plane, iref):
        m = (iref[...] != PAD_ID).astype(jnp.float32)
        cnt = jnp.sum(m, axis=1, keepdims=True)
        return feat[plane] / (cnt + 1e-8)

    ge = mean(2, g_ref)
    we = mean(3, w_ref)
    de = mean(4, dd_ref)
    embed = jnp.concatenate([feat[0], feat[1], ge, we, de, feat[5]], axis=1)

    linear = bias_ref[0, 0] + jnp.sum(fct_ref[...], axis=1, keepdims=True)
    s = jnp.sum(embed, axis=1, keepdims=True)
    ss = jnp.sum(embed * embed, axis=1, keepdims=True)
    fm = linear + 0.5 * (s * s - ss)

    h = jnp.maximum(
        jnp.dot(embed, W1_ref[...], preferred_element_type=jnp.float32)
        + b1_ref[...], 0.0)
    h = jnp.maximum(
        jnp.dot(h, W2_ref[...], preferred_element_type=jnp.float32)
        + b2_ref[...], 0.0)
    h = jnp.maximum(
        jnp.dot(h, W3_ref[...], preferred_element_type=jnp.float32)
        + b3_ref[...], 0.0)
    y = jnp.dot(h, W4_ref[...], preferred_element_type=jnp.float32) + b4_ref[...]

    out_ref[...] = jax.nn.sigmoid(fm + y)


@functools.lru_cache(maxsize=None)
def _make_tc_kernel(interpret=False):
    grid = (B_TOT // BLK,)
    return pl.pallas_call(
        _tc_body,
        grid=grid,
        in_specs=[
            pl.BlockSpec((6, BLK, D), lambda i: (0, i, 0)),
            pl.BlockSpec((BLK, LF), lambda i: (i, 0)),
            pl.BlockSpec((BLK, LF), lambda i: (i, 0)),
            pl.BlockSpec((BLK, LF), lambda i: (i, 0)),
            pl.BlockSpec((BLK, 3), lambda i: (i, 0)),
            pl.BlockSpec((1, 1), lambda i: (0, 0)),
            pl.BlockSpec((6 * D, 256), lambda i: (0, 0)),
            pl.BlockSpec((1, 256), lambda i: (0, 0)),
            pl.BlockSpec((256, 128), lambda i: (0, 0)),
            pl.BlockSpec((1, 128), lambda i: (0, 0)),
            pl.BlockSpec((128, 64), lambda i: (0, 0)),
            pl.BlockSpec((1, 64), lambda i: (0, 0)),
            pl.BlockSpec((64, 1), lambda i: (0, 0)),
            pl.BlockSpec((1, 1), lambda i: (0, 0)),
        ],
        out_specs=pl.BlockSpec((BLK, 1), lambda i: (i, 0)),
        out_shape=jax.ShapeDtypeStruct((B_TOT, 1), jnp.float32),
        interpret=interpret,
    )


def kernel(user, item, genres, writers, directors, year,
           emb_table, fc_table, bias,
           W1, b1, W2, b2, W3, b3, W4, b4):
    user = user.astype(jnp.int32)
    item = item.astype(jnp.int32)
    year = year.astype(jnp.int32)
    genres = genres.astype(jnp.int32)
    writers = writers.astype(jnp.int32)
    directors = directors.astype(jnp.int32)

    uiy = (jnp.stack([user, item, year])
           .reshape(3, NW, UIY_OPS, 128)
           .transpose(1, 0, 2, 3)
           .reshape(NW * 3 * UIY_OPS, 128))
    pool_idx = jnp.stack([genres, writers, directors]).reshape(
        3, B_TOT * LF // IPO, IPO)

    fcw = jnp.pad(fc_table.reshape(-1), (0, (-fc_table.size) % 16)).reshape(-1, 16)

    feat, fcv = _make_sc_kernel()(emb_table, fcw, uiy, pool_idx)
    fct = fcv.reshape(NW, 3, EPT).transpose(0, 2, 1).reshape(B_TOT, 3)

    out = _make_tc_kernel()(
        feat, genres, writers, directors, fct,
        bias.reshape(1, 1),
        W1, b1.reshape(1, 256), W2, b2.reshape(1, 128),
        W3, b3.reshape(1, 64), W4, b4.reshape(1, 1))
    return out.reshape(B_TOT)
